# Initial kernel scaffold; baseline (speedup 1.0000x reference)
import jax
import jax.numpy as jnp
from jax.experimental import pallas as pl


def kernel(x, w):
    raise NotImplementedError("write your pallas kernel here")


# speedup vs baseline: 2.4999x; 2.4999x over previous
import functools

import jax
import jax.numpy as jnp
from jax.experimental import pallas as pl
from jax.experimental.pallas import tpu as pltpu


def _round_up(a, b):
    return (a + b - 1) // b * b


def _ds_ncl_kernel(x_ref, w_ref, o_ref, xt_ref, *, slope, TO):
    xt_ref[...] = x_ref[0].T
    even_t = xt_ref[pl.Slice(0, TO, 2), :]
    odd_t = xt_ref[pl.Slice(1, TO, 2), :]
    y_t = jnp.dot(even_t, w_ref[0], preferred_element_type=jnp.float32)
    y_t += jnp.dot(odd_t, w_ref[1], preferred_element_type=jnp.float32)
    y_t = jnp.where(y_t > 0, y_t, slope * y_t)
    o_ref[0] = y_t.T.astype(o_ref.dtype)


def kernel(x, w, *, slope=0.01):
    B, C, L = x.shape
    assert w.shape == (C, C, 2), w.shape
    Lout = L // 2
    x = x[:, :, :2 * Lout]

    TO = min(512, _round_up(Lout, 8))
    Lp = _round_up(Lout, TO)
    if Lp != Lout:
        x = jnp.pad(x, ((0, 0), (0, 0), (0, 2 * (Lp - Lout))))

    w_t = jnp.transpose(w, (2, 1, 0))

    y = pl.pallas_call(
        functools.partial(_ds_ncl_kernel, slope=slope, TO=TO),
        out_shape=jax.ShapeDtypeStruct((B, C, Lp), x.dtype),
        grid=(B, Lp // TO),
        in_specs=[pl.BlockSpec((1, C, 2 * TO), lambda b, j: (b, 0, j)),
                  pl.BlockSpec((2, C, C), lambda b, j: (0, 0, 0))],
        out_specs=pl.BlockSpec((1, C, TO), lambda b, j: (b, 0, j)),
        scratch_shapes=[pltpu.VMEM((2 * TO, C), jnp.float32)],
        compiler_params=pltpu.CompilerParams(
            dimension_semantics=("parallel", "parallel"),
            vmem_limit_bytes=64 * 1024 * 1024),
    )(x, w_t)

    if Lp != Lout:
        y = y[:, :, :Lout]
    return y


# speedup vs baseline: 3.6923x; 1.4770x over previous
import functools

import jax
import jax.numpy as jnp
from jax.experimental import pallas as pl
from jax.experimental.pallas import tpu as pltpu


def _round_up(a, b):
    return (a + b - 1) // b * b


def _ds_ncl_kernel(x_ref, w_ref, o_ref, xt_ref, *, slope, TO):
    xt_ref[...] = x_ref[0].T
    even_t = xt_ref[pl.Slice(0, TO, 2), :]
    odd_t = xt_ref[pl.Slice(1, TO, 2), :]
    y_t = jnp.dot(even_t, w_ref[0], preferred_element_type=jnp.float32)
    y_t += jnp.dot(odd_t, w_ref[1], preferred_element_type=jnp.float32)
    y_t = jnp.where(y_t > 0, y_t, slope * y_t)
    o_ref[0] = y_t.T.astype(o_ref.dtype)


def kernel(x, w, *, slope=0.01):
    B, C, L = x.shape
    assert w.shape == (C, C, 2), w.shape
    Lout = L // 2
    x = x[:, :, :2 * Lout]

    TO = min(1024, _round_up(Lout, 8))
    Lp = _round_up(Lout, TO)
    if Lp != Lout:
        x = jnp.pad(x, ((0, 0), (0, 0), (0, 2 * (Lp - Lout))))

    w_t = jnp.transpose(w, (2, 1, 0))

    y = pl.pallas_call(
        functools.partial(_ds_ncl_kernel, slope=slope, TO=TO),
        out_shape=jax.ShapeDtypeStruct((B, C, Lp), x.dtype),
        grid=(B, Lp // TO),
        in_specs=[pl.BlockSpec((1, C, 2 * TO), lambda b, j: (b, 0, j)),
                  pl.BlockSpec((2, C, C), lambda b, j: (0, 0, 0))],
        out_specs=pl.BlockSpec((1, C, TO), lambda b, j: (b, 0, j)),
        scratch_shapes=[pltpu.VMEM((2 * TO, C), jnp.float32)],
        compiler_params=pltpu.CompilerParams(
            dimension_semantics=("parallel", "parallel"),
            vmem_limit_bytes=64 * 1024 * 1024),
    )(x, w_t)

    if Lp != Lout:
        y = y[:, :, :Lout]
    return y


# speedup vs baseline: 4.7974x; 1.2993x over previous
import functools

import jax
import jax.numpy as jnp
from jax.experimental import pallas as pl
from jax.experimental.pallas import tpu as pltpu


def _round_up(a, b):
    return (a + b - 1) // b * b


def _ds_ncl_kernel(x_ref, w_ref, o_ref, xt_ref, *, slope, TO):
    xt_ref[...] = x_ref[0].T
    even_t = xt_ref[pl.Slice(0, TO, 2), :]
    odd_t = xt_ref[pl.Slice(1, TO, 2), :]
    y_t = jnp.dot(even_t, w_ref[0], preferred_element_type=jnp.float32)
    y_t += jnp.dot(odd_t, w_ref[1], preferred_element_type=jnp.float32)
    y_t = jnp.where(y_t > 0, y_t, slope * y_t)
    o_ref[0] = y_t.T.astype(o_ref.dtype)


def kernel(x, w, *, slope=0.01):
    B, C, L = x.shape
    assert w.shape == (C, C, 2), w.shape
    Lout = L // 2
    x = x[:, :, :2 * Lout]

    TO = min(2048, _round_up(Lout, 8))
    Lp = _round_up(Lout, TO)
    if Lp != Lout:
        x = jnp.pad(x, ((0, 0), (0, 0), (0, 2 * (Lp - Lout))))

    w_t = jnp.transpose(w, (2, 1, 0))

    y = pl.pallas_call(
        functools.partial(_ds_ncl_kernel, slope=slope, TO=TO),
        out_shape=jax.ShapeDtypeStruct((B, C, Lp), x.dtype),
        grid=(B, Lp // TO),
        in_specs=[pl.BlockSpec((1, C, 2 * TO), lambda b, j: (b, 0, j)),
                  pl.BlockSpec((2, C, C), lambda b, j: (0, 0, 0))],
        out_specs=pl.BlockSpec((1, C, TO), lambda b, j: (b, 0, j)),
        scratch_shapes=[pltpu.VMEM((2 * TO, C), jnp.float32)],
        compiler_params=pltpu.CompilerParams(
            dimension_semantics=("parallel", "parallel"),
            vmem_limit_bytes=64 * 1024 * 1024),
    )(x, w_t)

    if Lp != Lout:
        y = y[:, :, :Lout]
    return y
